# baseline (speedup 1.0000x reference)
import functools

import jax
import jax.numpy as jnp
from jax import lax
from jax.experimental import pallas as pl
from jax.experimental.pallas import tpu as pltpu
from jax.experimental.pallas import tpu_sc as plsc

NC = 2
NS = 16
NW = NC * NS
LANES = 16
CE = 112


def _mesh():
  return plsc.VectorSubcoreMesh(
      core_axis_name="c", subcore_axis_name="s", num_cores=NC,
      num_subcores=NS)


def _make_sc_deg(CH, NROW):
  sl_n = NROW // NS

  @functools.partial(
      pl.kernel, mesh=_mesh(),
      out_type=jax.ShapeDtypeStruct((NC, NROW), jnp.float32),
      scratch_types=[
          pltpu.VMEM((CH, CE), jnp.int32),
          pltpu.VMEM((CH, CE), jnp.float32),
          pltpu.VMEM((sl_n,), jnp.float32),
          pltpu.VMEM_SHARED((NROW,), jnp.float32),
      ],
      name="sc_deg",
  )
  def sc_deg(d_hbm, w_hbm, out_hbm, didx, wv, zv, deg_s):
    c = lax.axis_index("c")
    s = lax.axis_index("s")
    wid = s * NC + c
    z16 = jnp.zeros((LANES,), jnp.float32)

    def zb(i, carry):
      off = jnp.minimum(i * LANES, sl_n - LANES)
      zv[pl.ds(off, LANES)] = z16
      return carry
    lax.fori_loop(0, -(-sl_n // LANES), zb, 0)
    pltpu.sync_copy(zv, deg_s.at[pl.ds(s * sl_n, sl_n)])
    pltpu.sync_copy(d_hbm.at[wid], didx)
    pltpu.sync_copy(w_hbm.at[wid], wv)
    plsc.subcore_barrier()

    def ch(j, carry):
      pltpu.sync_copy(wv.at[j], deg_s.at[didx.at[j]], add=True)
      return carry
    lax.fori_loop(0, CH, ch, 0)
    plsc.subcore_barrier()
    pltpu.sync_copy(deg_s.at[pl.ds(s * sl_n, sl_n)],
                    out_hbm.at[c].at[pl.ds(s * sl_n, sl_n)])

  return sc_deg


def _make_sc_agg(CH, NROW, Wd):
  sl_n = NROW // NS
  KW = Wd // LANES
  assert CH % 3 == 0

  @functools.partial(
      pl.kernel, mesh=_mesh(),
      out_type=jax.ShapeDtypeStruct((NC, NROW, Wd), jnp.float32),
      scratch_types=[
          pltpu.VMEM((8, CE), jnp.int32),
          pltpu.VMEM((8, CE), jnp.int32),
          pltpu.VMEM((8, CE), jnp.float32),
          pltpu.VMEM((3, CE, Wd), jnp.float32),
          pltpu.SemaphoreType.DMA,
          pltpu.SemaphoreType.DMA,
          pltpu.SemaphoreType.DMA,
          pltpu.SemaphoreType.DMA,
          pltpu.SemaphoreType.DMA,
          pltpu.SemaphoreType.DMA,
          pltpu.SemaphoreType.DMA,
          pltpu.VMEM_SHARED((NROW, Wd), jnp.float32),
      ],
      name="sc_agg",
  )
  def sc_agg(hw_hbm, s_hbm, d_hbm, n_hbm, out_hbm,
             sidx, didx, nv, rows, semi,
             semg0, semg1, semg2, sems0, sems1, sems2, acc_s):
    c = lax.axis_index("c")
    s = lax.axis_index("s")
    wid = s * NC + c
    z16 = jnp.zeros((LANES,), jnp.float32)
    semg = (semg0, semg1, semg2)
    sems = (sems0, sems1, sems2)

    def zr(i, carry):
      for k in range(KW):
        rows[0, i, pl.ds(k * LANES, LANES)] = z16
      return carry
    lax.fori_loop(0, CE, zr, 0)
    left = sl_n
    while left > 0:
      q = min(left, CE)
      pltpu.sync_copy(rows.at[0].at[pl.ds(0, q)],
                      acc_s.at[pl.ds(s * sl_n + (sl_n - left), q)])
      left -= q
    plsc.subcore_barrier()

    def fire_idx(j):
      slot = j & 7
      pltpu.async_copy(s_hbm.at[wid].at[j], sidx.at[slot], semi)
      pltpu.async_copy(d_hbm.at[wid].at[j], didx.at[slot], semi)
      pltpu.async_copy(n_hbm.at[wid].at[j], nv.at[slot], semi)

    def wait_idx(j):
      slot = j & 7
      pltpu.make_async_copy(s_hbm.at[wid].at[j], sidx.at[slot], semi).wait()
      pltpu.make_async_copy(d_hbm.at[wid].at[j], didx.at[slot], semi).wait()
      pltpu.make_async_copy(n_hbm.at[wid].at[j], nv.at[slot], semi).wait()

    def fire_gather(j, b):
      pltpu.async_copy(hw_hbm.at[sidx.at[j & 7]], rows.at[b], semg[b])

    def wait_scatter(j, b):
      pltpu.make_async_copy(rows.at[b], acc_s.at[didx.at[j & 7]],
                            sems[b]).wait()

    for jj in range(8):
      fire_idx(jnp.int32(jj))
    wait_idx(jnp.int32(0))
    fire_gather(jnp.int32(0), 0)
    wait_idx(jnp.int32(1))
    fire_gather(jnp.int32(1), 1)

    def process(j, b, bn):
      slot = j & 7
      pltpu.make_async_copy(hw_hbm.at[sidx.at[slot]], rows.at[b],
                            semg[b]).wait()

      def se(g, carry):
        n16 = nv[slot, pl.ds(g * LANES, LANES)]
        for l in range(LANES):
          e = g * LANES + l
          nrm = n16[l]
          for k in range(KW):
            sl = pl.ds(k * LANES, LANES)
            rows[b, e, sl] = rows[b, e, sl] * nrm
        return carry
      lax.fori_loop(0, CE // LANES, se, 0)
      pltpu.async_copy(rows.at[b], acc_s.at[didx.at[slot]], sems[b],
                       add=True)

      @pl.when(j + 2 < CH)
      def _():
        @pl.when(j >= 1)
        def _():
          wait_scatter(j - 1, bn)

          @pl.when(j + 7 < CH)
          def _():
            fire_idx(j + 7)
        wait_idx(j + 2)
        fire_gather(j + 2, bn)

    def step(m, carry):
      j = 3 * m
      process(j, 0, 2)
      process(j + 1, 1, 0)
      process(j + 2, 2, 1)
      return carry
    lax.fori_loop(0, CH // 3, step, 0)
    wait_scatter(jnp.int32(CH - 3), (CH - 3) % 3)
    wait_scatter(jnp.int32(CH - 2), (CH - 2) % 3)
    wait_scatter(jnp.int32(CH - 1), (CH - 1) % 3)

    plsc.subcore_barrier()
    base = s * sl_n
    pltpu.sync_copy(acc_s.at[pl.ds(base, sl_n)],
                    out_hbm.at[c].at[pl.ds(base, sl_n)])

  return sc_agg


def _tc1(deg2, xp, W1, N, G):
  R = N // G
  D = xp.shape[1]
  H = W1.shape[1]

  def body(deg_ref, x_ref, w_ref, dinv_ref, hw_ref):
    deg = deg_ref[0] + deg_ref[1]
    dinv = jnp.where(deg > 0, lax.rsqrt(jnp.maximum(deg, 1e-12)), 0.0)
    dinv_ref[...] = dinv
    hw = jnp.dot(x_ref[...], w_ref[...], preferred_element_type=jnp.float32)
    hw_ref[...] = hw * dinv

  return pl.pallas_call(
      body,
      grid=(G,),
      in_specs=[
          pl.BlockSpec((2, R, 1), lambda i: (0, i, 0)),
          pl.BlockSpec((R, D), lambda i: (i, 0)),
          pl.BlockSpec((D, H), lambda i: (0, 0)),
      ],
      out_specs=[
          pl.BlockSpec((R, 1), lambda i: (i, 0)),
          pl.BlockSpec((R, H), lambda i: (i, 0)),
      ],
      out_shape=[
          jax.ShapeDtypeStruct((N, 1), jnp.float32),
          jax.ShapeDtypeStruct((N, H), jnp.float32),
      ],
  )(deg2, xp, W1)


def _tc_layer(acc, dinvc, b, W, N, G):
  R = N // G
  H = acc.shape[2]
  Hn = W.shape[1]

  def body(acc_ref, dinv_ref, b_ref, w_ref, out_ref):
    dinv = dinv_ref[...]
    h = jnp.maximum((acc_ref[0] + acc_ref[1]) * dinv + b_ref[...], 0.0)
    out_ref[...] = jnp.dot(h, w_ref[...],
                           preferred_element_type=jnp.float32) * dinv

  return pl.pallas_call(
      body,
      grid=(G,),
      in_specs=[
          pl.BlockSpec((2, R, H), lambda i: (0, i, 0)),
          pl.BlockSpec((R, 1), lambda i: (i, 0)),
          pl.BlockSpec((1, H), lambda i: (0, 0)),
          pl.BlockSpec((H, Hn), lambda i: (0, 0)),
      ],
      out_specs=pl.BlockSpec((R, Hn), lambda i: (i, 0)),
      out_shape=jax.ShapeDtypeStruct((N, Hn), jnp.float32),
  )(acc, dinvc, b, W)


def _tc_relu(acc, dinvc, b, N, G):
  R = N // G
  H = acc.shape[2]

  def body(acc_ref, dinv_ref, b_ref, out_ref):
    dinv = dinv_ref[...]
    h = jnp.maximum((acc_ref[0] + acc_ref[1]) * dinv + b_ref[...], 0.0)
    out_ref[...] = h * dinv

  return pl.pallas_call(
      body,
      grid=(G,),
      in_specs=[
          pl.BlockSpec((2, R, H), lambda i: (0, i, 0)),
          pl.BlockSpec((R, 1), lambda i: (i, 0)),
          pl.BlockSpec((1, H), lambda i: (0, 0)),
      ],
      out_specs=pl.BlockSpec((R, H), lambda i: (i, 0)),
      out_shape=jax.ShapeDtypeStruct((N, H), jnp.float32),
  )(acc, dinvc, b)


def _tc_final(acc, dinvc, W3, b3, N, G):
  R = N // G
  H = acc.shape[2]
  C = W3.shape[1]

  def body(acc_ref, dinv_ref, w_ref, b_ref, out_ref):
    agg = (acc_ref[0] + acc_ref[1]) * dinv_ref[...]
    out_ref[...] = jnp.dot(agg, w_ref[...],
                           preferred_element_type=jnp.float32) + b_ref[...]

  return pl.pallas_call(
      body,
      grid=(G,),
      in_specs=[
          pl.BlockSpec((2, R, H), lambda i: (0, i, 0)),
          pl.BlockSpec((R, 1), lambda i: (i, 0)),
          pl.BlockSpec((H, C), lambda i: (0, 0)),
          pl.BlockSpec((1, C), lambda i: (0, 0)),
      ],
      out_specs=pl.BlockSpec((R, C), lambda i: (i, 0)),
      out_shape=jax.ShapeDtypeStruct((N, C), jnp.float32),
  )(acc, dinvc, W3, b3)


def kernel(x, edge_index, edge_weight, W1, b1, W2, b2, W3, b3):
  N, D = x.shape
  E = edge_index.shape[1]
  H = W1.shape[1]
  C = W3.shape[1]

  E_tot = E + N
  CH = -(-E_tot // (NW * CE))
  CH = -(-CH // 3) * 3
  E_pad = NW * CH * CE

  src = edge_index[0]
  dst = edge_index[1]
  loop = jnp.arange(N, dtype=src.dtype)
  pad = E_pad - E_tot
  spread = jnp.arange(pad, dtype=src.dtype) % jnp.int32(N)
  sE = jnp.concatenate([src, loop, spread])
  dE = jnp.concatenate([dst, loop, spread])
  wE = jnp.concatenate([edge_weight, jnp.ones((N,), jnp.float32),
                        jnp.zeros((pad,), jnp.float32)])
  sE3 = sE.reshape(NW, CH, CE)
  dE3 = dE.reshape(NW, CH, CE)
  wE3 = wE.reshape(NW, CH, CE)

  NROW = -(-N // (NS * 8)) * (NS * 8)
  xp = jnp.pad(x, ((0, NROW - N), (0, 0)))
  G = 8

  Npad = -(-N // 2048) * 2048
  deg2 = _make_sc_deg(CH, Npad)(dE3, wE3)
  dinvc, hw1 = _tc1(deg2[:, :NROW].reshape(2, NROW, 1), xp, W1, NROW, G)

  agg = _make_sc_agg(CH, NROW, H)
  acc1 = agg(hw1, sE3, dE3, wE3)
  hw2 = _tc_layer(acc1, dinvc, b1.reshape(1, H), W2, NROW, G)
  acc2 = agg(hw2, sE3, dE3, wE3)
  h3 = _tc_relu(acc2, dinvc, b2.reshape(1, H), NROW, G)
  acc3 = agg(h3, sE3, dE3, wE3)
  out = _tc_final(acc3, dinvc, W3, b3.reshape(1, C), NROW, G)
  return out[:N]


# speedup vs baseline: 1.0203x; 1.0203x over previous
import functools

import jax
import jax.numpy as jnp
from jax import lax
from jax.experimental import pallas as pl
from jax.experimental.pallas import tpu as pltpu
from jax.experimental.pallas import tpu_sc as plsc

NC = 2
NS = 16
NW = NC * NS
LANES = 16
CE = 112


def _mesh():
  return plsc.VectorSubcoreMesh(
      core_axis_name="c", subcore_axis_name="s", num_cores=NC,
      num_subcores=NS)


def _make_sc_deg(CH, NROW):
  sl_n = NROW // NS

  @functools.partial(
      pl.kernel, mesh=_mesh(),
      out_type=jax.ShapeDtypeStruct((NC, NROW), jnp.float32),
      scratch_types=[
          pltpu.VMEM((CH, CE), jnp.int32),
          pltpu.VMEM((CH, CE), jnp.float32),
          pltpu.VMEM((sl_n,), jnp.float32),
          pltpu.VMEM_SHARED((NROW,), jnp.float32),
      ],
      name="sc_deg",
  )
  def sc_deg(d_hbm, w_hbm, out_hbm, didx, wv, zv, deg_s):
    c = lax.axis_index("c")
    s = lax.axis_index("s")
    wid = s * NC + c
    z16 = jnp.zeros((LANES,), jnp.float32)

    def zb(i, carry):
      off = jnp.minimum(i * LANES, sl_n - LANES)
      zv[pl.ds(off, LANES)] = z16
      return carry
    lax.fori_loop(0, -(-sl_n // LANES), zb, 0)
    pltpu.sync_copy(zv, deg_s.at[pl.ds(s * sl_n, sl_n)])
    pltpu.sync_copy(d_hbm.at[wid], didx)
    pltpu.sync_copy(w_hbm.at[wid], wv)
    plsc.subcore_barrier()

    def ch(j, carry):
      pltpu.sync_copy(wv.at[j], deg_s.at[didx.at[j]], add=True)
      return carry
    lax.fori_loop(0, CH, ch, 0)
    plsc.subcore_barrier()
    pltpu.sync_copy(deg_s.at[pl.ds(s * sl_n, sl_n)],
                    out_hbm.at[c].at[pl.ds(s * sl_n, sl_n)])

  return sc_deg


def _make_sc_agg(CH, NROW, Wd):
  sl_n = NROW // NS
  KW = Wd // LANES
  assert CH % 3 == 0

  @functools.partial(
      pl.kernel, mesh=_mesh(),
      out_type=jax.ShapeDtypeStruct((NC, NROW, Wd), jnp.float32),
      scratch_types=[
          pltpu.VMEM((8, CE), jnp.int32),
          pltpu.VMEM((8, CE), jnp.int32),
          pltpu.VMEM((8, CE), jnp.float32),
          pltpu.VMEM((3, CE, Wd), jnp.float32),
          pltpu.SemaphoreType.DMA,
          pltpu.SemaphoreType.DMA,
          pltpu.SemaphoreType.DMA,
          pltpu.SemaphoreType.DMA,
          pltpu.SemaphoreType.DMA,
          pltpu.SemaphoreType.DMA,
          pltpu.SemaphoreType.DMA,
          pltpu.VMEM_SHARED((NROW, Wd), jnp.float32),
      ],
      name="sc_agg",
  )
  def sc_agg(hw_hbm, s_hbm, d_hbm, n_hbm, out_hbm,
             sidx, didx, nv, rows, semi,
             semg0, semg1, semg2, sems0, sems1, sems2, acc_s):
    c = lax.axis_index("c")
    s = lax.axis_index("s")
    wid = s * NC + c
    z16 = jnp.zeros((LANES,), jnp.float32)
    semg = (semg0, semg1, semg2)
    sems = (sems0, sems1, sems2)

    def zr(i, carry):
      for k in range(KW):
        rows[0, i, pl.ds(k * LANES, LANES)] = z16
      return carry
    lax.fori_loop(0, CE, zr, 0)
    left = sl_n
    while left > 0:
      q = min(left, CE)
      pltpu.sync_copy(rows.at[0].at[pl.ds(0, q)],
                      acc_s.at[pl.ds(s * sl_n + (sl_n - left), q)])
      left -= q
    plsc.subcore_barrier()

    def fire_idx(j):
      slot = j & 7
      pltpu.async_copy(s_hbm.at[wid].at[j], sidx.at[slot], semi)
      pltpu.async_copy(d_hbm.at[wid].at[j], didx.at[slot], semi)
      pltpu.async_copy(n_hbm.at[wid].at[j], nv.at[slot], semi)

    def wait_idx(j):
      slot = j & 7
      pltpu.make_async_copy(s_hbm.at[wid].at[j], sidx.at[slot], semi).wait()
      pltpu.make_async_copy(d_hbm.at[wid].at[j], didx.at[slot], semi).wait()
      pltpu.make_async_copy(n_hbm.at[wid].at[j], nv.at[slot], semi).wait()

    def fire_gather(j, b):
      pltpu.async_copy(hw_hbm.at[sidx.at[j & 7]], rows.at[b], semg[b])

    def wait_scatter(j, b):
      pltpu.make_async_copy(rows.at[b], acc_s.at[didx.at[j & 7]],
                            sems[b]).wait()

    for jj in range(8):
      fire_idx(jnp.int32(jj))
    wait_idx(jnp.int32(0))
    fire_gather(jnp.int32(0), 0)
    wait_idx(jnp.int32(1))
    fire_gather(jnp.int32(1), 1)

    def process(j, b, bn):
      slot = j & 7
      pltpu.make_async_copy(hw_hbm.at[sidx.at[slot]], rows.at[b],
                            semg[b]).wait()

      def se(g, carry):
        n16 = nv[slot, pl.ds(g * LANES, LANES)]
        for l in range(LANES):
          e = g * LANES + l
          nrm = n16[l]
          for k in range(KW):
            sl = pl.ds(k * LANES, LANES)
            rows[b, e, sl] = rows[b, e, sl] * nrm
        return carry
      lax.fori_loop(0, CE // LANES, se, 0)
      pltpu.async_copy(rows.at[b], acc_s.at[didx.at[slot]], sems[b],
                       add=True)

      @pl.when(j + 2 < CH)
      def _():
        @pl.when(j >= 1)
        def _():
          wait_scatter(j - 1, bn)

          @pl.when(j + 7 < CH)
          def _():
            fire_idx(j + 7)
        wait_idx(j + 2)
        fire_gather(j + 2, bn)

    def step(m, carry):
      j = 3 * m
      process(j, 0, 2)
      process(j + 1, 1, 0)
      process(j + 2, 2, 1)
      return carry
    lax.fori_loop(0, CH // 3, step, 0)
    wait_scatter(jnp.int32(CH - 3), (CH - 3) % 3)
    wait_scatter(jnp.int32(CH - 2), (CH - 2) % 3)
    wait_scatter(jnp.int32(CH - 1), (CH - 1) % 3)

    plsc.subcore_barrier()
    base = s * sl_n
    pltpu.sync_copy(acc_s.at[pl.ds(base, sl_n)],
                    out_hbm.at[c].at[pl.ds(base, sl_n)])

  return sc_agg


def _tc1(deg2, xp, W1, N, G):
  R = N // G
  D = xp.shape[1]
  H = W1.shape[1]

  def body(deg_ref, x_ref, w_ref, dinv_ref, hw_ref):
    deg = deg_ref[0] + deg_ref[1]
    dinv = jnp.where(deg > 0, lax.rsqrt(jnp.maximum(deg, 1e-12)), 0.0)
    dinv_ref[...] = dinv
    hw = jnp.dot(x_ref[...], w_ref[...], preferred_element_type=jnp.float32)
    hw_ref[...] = hw * dinv

  return pl.pallas_call(
      body,
      grid=(G,),
      in_specs=[
          pl.BlockSpec((2, R, 1), lambda i: (0, i, 0)),
          pl.BlockSpec((R, D), lambda i: (i, 0)),
          pl.BlockSpec((D, H), lambda i: (0, 0)),
      ],
      out_specs=[
          pl.BlockSpec((R, 1), lambda i: (i, 0)),
          pl.BlockSpec((R, H), lambda i: (i, 0)),
      ],
      out_shape=[
          jax.ShapeDtypeStruct((N, 1), jnp.float32),
          jax.ShapeDtypeStruct((N, H), jnp.float32),
      ],
  )(deg2, xp, W1)


def _tc_layer(acc, dinvc, b, W, N, G):
  R = N // G
  H = acc.shape[2]
  Hn = W.shape[1]

  def body(acc_ref, dinv_ref, b_ref, w_ref, out_ref):
    dinv = dinv_ref[...]
    h = jnp.maximum((acc_ref[0] + acc_ref[1]) * dinv + b_ref[...], 0.0)
    out_ref[...] = jnp.dot(h, w_ref[...],
                           preferred_element_type=jnp.float32) * dinv

  return pl.pallas_call(
      body,
      grid=(G,),
      in_specs=[
          pl.BlockSpec((2, R, H), lambda i: (0, i, 0)),
          pl.BlockSpec((R, 1), lambda i: (i, 0)),
          pl.BlockSpec((1, H), lambda i: (0, 0)),
          pl.BlockSpec((H, Hn), lambda i: (0, 0)),
      ],
      out_specs=pl.BlockSpec((R, Hn), lambda i: (i, 0)),
      out_shape=jax.ShapeDtypeStruct((N, Hn), jnp.float32),
  )(acc, dinvc, b, W)


def _tc_relu(acc, dinvc, b, N, G):
  R = N // G
  H = acc.shape[2]

  def body(acc_ref, dinv_ref, b_ref, out_ref):
    dinv = dinv_ref[...]
    h = jnp.maximum((acc_ref[0] + acc_ref[1]) * dinv + b_ref[...], 0.0)
    out_ref[...] = h * dinv

  return pl.pallas_call(
      body,
      grid=(G,),
      in_specs=[
          pl.BlockSpec((2, R, H), lambda i: (0, i, 0)),
          pl.BlockSpec((R, 1), lambda i: (i, 0)),
          pl.BlockSpec((1, H), lambda i: (0, 0)),
      ],
      out_specs=pl.BlockSpec((R, H), lambda i: (i, 0)),
      out_shape=jax.ShapeDtypeStruct((N, H), jnp.float32),
  )(acc, dinvc, b)


def _tc_final(acc, dinvc, W3, b3, N, G):
  R = N // G
  H = acc.shape[2]
  C = W3.shape[1]

  def body(acc_ref, dinv_ref, w_ref, b_ref, out_ref):
    agg = (acc_ref[0] + acc_ref[1]) * dinv_ref[...]
    out_ref[...] = jnp.dot(agg, w_ref[...],
                           preferred_element_type=jnp.float32) + b_ref[...]

  return pl.pallas_call(
      body,
      grid=(G,),
      in_specs=[
          pl.BlockSpec((2, R, H), lambda i: (0, i, 0)),
          pl.BlockSpec((R, 1), lambda i: (i, 0)),
          pl.BlockSpec((H, C), lambda i: (0, 0)),
          pl.BlockSpec((1, C), lambda i: (0, 0)),
      ],
      out_specs=pl.BlockSpec((R, C), lambda i: (i, 0)),
      out_shape=jax.ShapeDtypeStruct((N, C), jnp.float32),
  )(acc, dinvc, W3, b3)


def kernel(x, edge_index, edge_weight, W1, b1, W2, b2, W3, b3):
  N, D = x.shape
  E = edge_index.shape[1]
  H = W1.shape[1]
  C = W3.shape[1]

  E_tot = E + N
  CH = -(-E_tot // (NW * CE))
  CH = -(-CH // 3) * 3
  E_pad = NW * CH * CE

  src = edge_index[0]
  dst = edge_index[1]
  loop = jnp.arange(N, dtype=src.dtype)
  pad = E_pad - E_tot
  spread = jnp.arange(pad, dtype=src.dtype) % jnp.int32(N)
  sE = jnp.concatenate([src, loop, spread])
  dE = jnp.concatenate([dst, loop, spread])
  wE = jnp.concatenate([edge_weight, jnp.ones((N,), jnp.float32),
                        jnp.zeros((pad,), jnp.float32)])
  sE3 = sE.reshape(NW, CH, CE)
  dE3 = dE.reshape(NW, CH, CE)
  wE3 = wE.reshape(NW, CH, CE)

  NROW = -(-N // (NS * 8)) * (NS * 8)
  xp = jnp.pad(x, ((0, NROW - N), (0, 0)))
  G = 4

  Npad = -(-N // 2048) * 2048
  deg2 = _make_sc_deg(CH, Npad)(dE3, wE3)
  dinvc, hw1 = _tc1(deg2[:, :NROW].reshape(2, NROW, 1), xp, W1, NROW, G)

  agg = _make_sc_agg(CH, NROW, H)
  acc1 = agg(hw1, sE3, dE3, wE3)
  hw2 = _tc_layer(acc1, dinvc, b1.reshape(1, H), W2, NROW, G)
  acc2 = agg(hw2, sE3, dE3, wE3)
  h3 = _tc_relu(acc2, dinvc, b2.reshape(1, H), NROW, G)
  acc3 = agg(h3, sE3, dE3, wE3)
  out = _tc_final(acc3, dinvc, W3, b3.reshape(1, C), NROW, G)
  return out[:N]
